# baseline (speedup 1.0000x reference)
import functools

import jax
import jax.numpy as jnp
import numpy as np
from jax import lax
from jax.experimental import pallas as pl
from jax.experimental.pallas import tpu as pltpu
from jax.experimental.pallas import tpu_sc as plsc

F = 26
D = 16
B = 16384
VOCAB = 100000
R = F * VOCAB
BF = B * F
EMBED_OUT = F * D
H1, H2 = 128, 64
BN_EPS = 1e-5

NC, NS = 2, 16
NW = NC * NS
BW = B // NW
NSTEP = F * D


@functools.cache
def _make_sc_embed():
    mesh = plsc.VectorSubcoreMesh(core_axis_name="c", subcore_axis_name="s")

    @functools.partial(
        pl.kernel,
        mesh=mesh,
        out_type=jax.ShapeDtypeStruct((EMBED_OUT, B), jnp.float32),
        scratch_types=[
            pltpu.VMEM((BW,), jnp.int32),
            pltpu.VMEM((BW,), jnp.int32),
            pltpu.VMEM((BW,), jnp.int32),
            pltpu.VMEM((BW,), jnp.float32),
            pltpu.VMEM((BW,), jnp.float32),
            pltpu.SemaphoreType.DMA,
            pltpu.SemaphoreType.DMA,
            pltpu.SemaphoreType.DMA,
            pltpu.SemaphoreType.DMA,
        ],
        compiler_params=pltpu.CompilerParams(use_tc_tiling_on_sc=False),
    )
    def _sc_embed(idxt_hbm, emb_hbm, out_t,
                  idx_v, sidx_a, sidx_b, val_a, val_b,
                  gsem_a, gsem_b, wsem_a, wsem_b):
        wid = lax.axis_index("s") * NC + lax.axis_index("c")
        b0 = wid * BW
        sidx = (sidx_a, sidx_b)
        vals = (val_a, val_b)
        gsems = (gsem_a, gsem_b)
        wsems = (wsem_a, wsem_b)

        def build_and_fire(t):
            f = lax.shift_right_logical(t, 4)
            d = lax.bitwise_and(t, 15)
            p = lax.bitwise_and(t, 1)

            @pl.when(d == 0)
            def _():
                pltpu.sync_copy(idxt_hbm.at[pl.ds(f * B + b0, BW)], idx_v)

            shift = jnp.broadcast_to(d * R, (16,)).astype(jnp.int32)

            def addloop(j, carry):
                s = pl.ds(j * 16, 16)
                _ = carry

                @pl.when(p == 0)
                def _():
                    sidx[0][s] = idx_v[s] + shift

                @pl.when(p == 1)
                def _():
                    sidx[1][s] = idx_v[s] + shift

                return carry

            lax.fori_loop(0, BW // 16, addloop, 0)

            @pl.when(p == 0)
            def _():
                pltpu.async_copy(emb_hbm.at[sidx[0]], vals[0], gsems[0])

            @pl.when(p == 1)
            def _():
                pltpu.async_copy(emb_hbm.at[sidx[1]], vals[1], gsems[1])

        def land(t):
            f = lax.shift_right_logical(t, 4)
            d = lax.bitwise_and(t, 15)
            p = lax.bitwise_and(t, 1)
            row = d * F + f

            def fin(q):
                pltpu.make_async_copy(emb_hbm.at[sidx[q]], vals[q],
                                      gsems[q]).wait()
                pltpu.async_copy(vals[q], out_t.at[row, pl.ds(b0, BW)],
                                 wsems[q])

            @pl.when(p == 0)
            def _():
                fin(0)

            @pl.when(p == 1)
            def _():
                fin(1)

        def drain(t):
            f = lax.shift_right_logical(t, 4)
            d = lax.bitwise_and(t, 15)
            p = lax.bitwise_and(t, 1)
            row = d * F + f

            def fin(q):
                pltpu.make_async_copy(vals[q], out_t.at[row, pl.ds(b0, BW)],
                                      wsems[q]).wait()

            @pl.when(p == 0)
            def _():
                fin(0)

            @pl.when(p == 1)
            def _():
                fin(1)

        build_and_fire(jnp.int32(0))

        def step(t, carry):
            @pl.when(t >= 2)
            def _():
                drain(t - 2)

            build_and_fire(t)
            land(t - 1)
            return carry

        lax.fori_loop(1, NSTEP, step, 0)
        drain(NSTEP - 2)
        land(NSTEP - 1)
        drain(NSTEP - 1)

    return _sc_embed


LCHUNK = 1664
LROWS_W = BF // NW
LNCHUNK = LROWS_W // LCHUNK


@functools.cache
def _make_sc_linear():
    mesh = plsc.VectorSubcoreMesh(core_axis_name="c", subcore_axis_name="s")

    @functools.partial(
        pl.kernel,
        mesh=mesh,
        out_type=jax.ShapeDtypeStruct((BF,), jnp.float32),
        scratch_types=[
            pltpu.VMEM((LCHUNK,), jnp.int32),
            pltpu.VMEM((LCHUNK,), jnp.float32),
            pltpu.SemaphoreType.DMA,
        ],
        compiler_params=pltpu.CompilerParams(use_tc_tiling_on_sc=False),
    )
    def _sc_linear(idx_hbm, lin_hbm, out_lin, idx_v, vals_v, sem):
        wid = lax.axis_index("s") * NC + lax.axis_index("c")
        base0 = wid * LROWS_W

        def body(c, carry):
            base = base0 + c * LCHUNK
            pltpu.sync_copy(idx_hbm.at[pl.ds(base, LCHUNK)], idx_v)
            pltpu.async_copy(lin_hbm.at[idx_v], vals_v, sem).wait()
            pltpu.sync_copy(vals_v, out_lin.at[pl.ds(base, LCHUNK)])
            return carry

        lax.fori_loop(0, LNCHUNK, body, 0)

    return _sc_linear


BLK = 1024


def _tc_body(h_ref, lin_ref, w1_ref, b1_ref, w2_ref, b2_ref, smat_ref,
             w3c_ref, out_ref):
    ht = h_ref[...]
    lin = lin_ref[...]
    linear = jnp.sum(lin, axis=0)

    hh = jnp.sum(ht * ht, axis=0)
    dn = (((0,), (0,)), ((), ()))
    s = lax.dot_general(ht, smat_ref[...], dn,
                        preferred_element_type=jnp.float32)
    fm = 0.5 * (jnp.sum(s * s, axis=1) - hh)

    a1 = lax.dot_general(ht, w1_ref[...], dn,
                         preferred_element_type=jnp.float32)
    a1 = jnp.maximum(a1 + b1_ref[...], 0.0)
    a2 = jnp.dot(a1, w2_ref[...], preferred_element_type=jnp.float32)
    a2 = jnp.maximum(a2 + b2_ref[...], 0.0)
    mlp = jnp.sum(a2 * w3c_ref[...][:, :H2], axis=1) + w3c_ref[0, H2]
    out_ref[...] = linear + fm + mlp


def _tc_mlp(ht, lint, w1p, b1f, w2f, b2f, smatp, w3c):
    grid = (B // BLK,)
    return pl.pallas_call(
        _tc_body,
        grid=grid,
        in_specs=[
            pl.BlockSpec((EMBED_OUT, BLK), lambda i: (0, i)),
            pl.BlockSpec((F, BLK), lambda i: (0, i)),
            pl.BlockSpec((EMBED_OUT, H1), lambda i: (0, 0)),
            pl.BlockSpec((1, H1), lambda i: (0, 0)),
            pl.BlockSpec((H1, H2), lambda i: (0, 0)),
            pl.BlockSpec((1, H2), lambda i: (0, 0)),
            pl.BlockSpec((EMBED_OUT, D), lambda i: (0, 0)),
            pl.BlockSpec((1, H2 + 1), lambda i: (0, 0)),
        ],
        out_specs=pl.BlockSpec((BLK,), lambda i: (i,)),
        out_shape=jax.ShapeDtypeStruct((B,), jnp.float32),
    )(ht, lint, w1p, b1f, w2f, b2f, smatp, w3c)


def kernel(x, emb_table, lin_w, lin_b, W1, b1, g1, be1, W2, b2, g2, be2,
           W3, b3):
    offs = jnp.arange(F, dtype=x.dtype) * VOCAB
    idxt = (jnp.transpose(x) + offs[:, None]).reshape(-1)
    emb_flat = jnp.transpose(emb_table).reshape(-1)
    lin_flat = jnp.transpose(lin_w).reshape(-1)

    ht = _make_sc_embed()(idxt, emb_flat)
    lin_vals = _make_sc_linear()(idxt, lin_flat)
    lint = lin_vals.reshape(F, B)

    bn = 1.0 / np.sqrt(1.0 + BN_EPS)
    w1f = W1 * (bn * g1)[None, :]
    w1p = w1f.reshape(F, D, H1).transpose(1, 0, 2).reshape(EMBED_OUT, H1)
    b1f = (b1 * bn * g1 + be1).reshape(1, H1)
    w2f = W2 * (bn * g2)[None, :]
    b2f = (b2 * bn * g2 + be2).reshape(1, H2)
    smatp = jnp.repeat(jnp.eye(D, dtype=jnp.float32), F, axis=0)
    w3c = jnp.concatenate([W3.reshape(1, H2), (lin_b + b3).reshape(1, 1)],
                          axis=1)
    return _tc_mlp(ht, lint, w1p, b1f, w2f, b2f, smatp, w3c)


# speedup vs baseline: 1.0354x; 1.0354x over previous
import functools

import jax
import jax.numpy as jnp
import numpy as np
from jax import lax
from jax.experimental import pallas as pl
from jax.experimental.pallas import tpu as pltpu
from jax.experimental.pallas import tpu_sc as plsc

F = 26
D = 16
B = 16384
VOCAB = 100000
R = F * VOCAB
BF = B * F
EMBED_OUT = F * D
H1, H2 = 128, 64
BN_EPS = 1e-5

NC, NS = 2, 16
NW = NC * NS
BW = B // NW
WROWS = F * BW


@functools.cache
def _make_sc_embed():
    mesh = plsc.VectorSubcoreMesh(core_axis_name="c", subcore_axis_name="s")

    @functools.partial(
        pl.kernel,
        mesh=mesh,
        out_type=jax.ShapeDtypeStruct((EMBED_OUT, B), jnp.float32),
        scratch_types=[
            pltpu.VMEM((WROWS,), jnp.int32),
            pltpu.VMEM((WROWS,), jnp.int32),
            pltpu.VMEM((WROWS,), jnp.int32),
            pltpu.VMEM((WROWS,), jnp.float32),
            pltpu.VMEM((WROWS,), jnp.float32),
            pltpu.SemaphoreType.DMA,
            pltpu.SemaphoreType.DMA,
            pltpu.SemaphoreType.DMA,
            pltpu.SemaphoreType.DMA,
        ],
        compiler_params=pltpu.CompilerParams(use_tc_tiling_on_sc=False),
    )
    def _sc_embed(idxw_hbm, emb_hbm, out_t,
                  idx_v, sidx_a, sidx_b, val_a, val_b,
                  gsem_a, gsem_b, wsem_a, wsem_b):
        wid = lax.axis_index("s") * NC + lax.axis_index("c")
        b0 = wid * BW
        sidx = (sidx_a, sidx_b)
        vals = (val_a, val_b)
        gsems = (gsem_a, gsem_b)
        wsems = (wsem_a, wsem_b)

        pltpu.sync_copy(idxw_hbm.at[pl.ds(wid * WROWS, WROWS)], idx_v)

        def build_and_fire(d, q):
            shift = jnp.broadcast_to(d * R, (16,)).astype(jnp.int32)

            def addloop(j, carry):
                s = pl.ds(j * 16, 16)
                sidx[q][s] = idx_v[s] + shift
                return carry

            lax.fori_loop(0, WROWS // 16, addloop, 0)
            pltpu.async_copy(emb_hbm.at[sidx[q]], vals[q], gsems[q])

        def land(d, q):
            pltpu.make_async_copy(emb_hbm.at[sidx[q]], vals[q],
                                  gsems[q]).wait()

            def wr(f, carry):
                pltpu.async_copy(vals[q].at[pl.ds(f * BW, BW)],
                                 out_t.at[d * F + f, pl.ds(b0, BW)],
                                 wsems[q])
                return carry

            lax.fori_loop(0, F, wr, 0)

        def drain(d, q):
            def wt(f, carry):
                pltpu.make_async_copy(vals[q].at[pl.ds(f * BW, BW)],
                                      out_t.at[d * F + f, pl.ds(b0, BW)],
                                      wsems[q]).wait()
                return carry

            lax.fori_loop(0, F, wt, 0)

        def even_odd(d, fn):
            p = lax.bitwise_and(d, 1)

            @pl.when(p == 0)
            def _():
                fn(d, 0)

            @pl.when(p == 1)
            def _():
                fn(d, 1)

        even_odd(jnp.int32(0), build_and_fire)

        def step(d, carry):
            @pl.when(d >= 2)
            def _():
                even_odd(d - 2, drain)

            even_odd(d, build_and_fire)
            even_odd(d - 1, land)
            return carry

        lax.fori_loop(1, D, step, 0)
        even_odd(jnp.int32(D - 2), drain)
        even_odd(jnp.int32(D - 1), land)
        even_odd(jnp.int32(D - 1), drain)

    return _sc_embed


LCHUNK = 1664
LROWS_W = BF // NW
LNCHUNK = LROWS_W // LCHUNK


@functools.cache
def _make_sc_linear():
    mesh = plsc.VectorSubcoreMesh(core_axis_name="c", subcore_axis_name="s")

    @functools.partial(
        pl.kernel,
        mesh=mesh,
        out_type=jax.ShapeDtypeStruct((BF,), jnp.float32),
        scratch_types=[
            pltpu.VMEM((LCHUNK,), jnp.int32),
            pltpu.VMEM((LCHUNK,), jnp.float32),
            pltpu.SemaphoreType.DMA,
        ],
        compiler_params=pltpu.CompilerParams(use_tc_tiling_on_sc=False),
    )
    def _sc_linear(idx_hbm, lin_hbm, out_lin, idx_v, vals_v, sem):
        wid = lax.axis_index("s") * NC + lax.axis_index("c")
        base0 = wid * LROWS_W

        def body(c, carry):
            base = base0 + c * LCHUNK
            pltpu.sync_copy(idx_hbm.at[pl.ds(base, LCHUNK)], idx_v)
            pltpu.async_copy(lin_hbm.at[idx_v], vals_v, sem).wait()
            pltpu.sync_copy(vals_v, out_lin.at[pl.ds(base, LCHUNK)])
            return carry

        lax.fori_loop(0, LNCHUNK, body, 0)

    return _sc_linear


BLK = 1024


def _tc_body(h_ref, lin_ref, w1_ref, b1_ref, w2_ref, b2_ref, smat_ref,
             w3c_ref, out_ref):
    ht = h_ref[...]
    lin = lin_ref[...]
    linear = jnp.sum(lin, axis=0)

    hh = jnp.sum(ht * ht, axis=0)
    dn = (((0,), (0,)), ((), ()))
    s = lax.dot_general(ht, smat_ref[...], dn,
                        preferred_element_type=jnp.float32)
    fm = 0.5 * (jnp.sum(s * s, axis=1) - hh)

    a1 = lax.dot_general(ht, w1_ref[...], dn,
                         preferred_element_type=jnp.float32)
    a1 = jnp.maximum(a1 + b1_ref[...], 0.0)
    a2 = jnp.dot(a1, w2_ref[...], preferred_element_type=jnp.float32)
    a2 = jnp.maximum(a2 + b2_ref[...], 0.0)
    mlp = jnp.sum(a2 * w3c_ref[...][:, :H2], axis=1) + w3c_ref[0, H2]
    out_ref[...] = linear + fm + mlp


def _tc_mlp(ht, lint, w1p, b1f, w2f, b2f, smatp, w3c):
    grid = (B // BLK,)
    return pl.pallas_call(
        _tc_body,
        grid=grid,
        in_specs=[
            pl.BlockSpec((EMBED_OUT, BLK), lambda i: (0, i)),
            pl.BlockSpec((F, BLK), lambda i: (0, i)),
            pl.BlockSpec((EMBED_OUT, H1), lambda i: (0, 0)),
            pl.BlockSpec((1, H1), lambda i: (0, 0)),
            pl.BlockSpec((H1, H2), lambda i: (0, 0)),
            pl.BlockSpec((1, H2), lambda i: (0, 0)),
            pl.BlockSpec((EMBED_OUT, D), lambda i: (0, 0)),
            pl.BlockSpec((1, H2 + 1), lambda i: (0, 0)),
        ],
        out_specs=pl.BlockSpec((BLK,), lambda i: (i,)),
        out_shape=jax.ShapeDtypeStruct((B,), jnp.float32),
    )(ht, lint, w1p, b1f, w2f, b2f, smatp, w3c)


def kernel(x, emb_table, lin_w, lin_b, W1, b1, g1, be1, W2, b2, g2, be2,
           W3, b3):
    offs = jnp.arange(F, dtype=x.dtype) * VOCAB
    idxt2 = jnp.transpose(x) + offs[:, None]
    idxt = idxt2.reshape(-1)
    idxw = idxt2.reshape(F, NW, BW).transpose(1, 0, 2).reshape(-1)
    emb_flat = jnp.transpose(emb_table).reshape(-1)
    lin_flat = jnp.transpose(lin_w).reshape(-1)

    ht = _make_sc_embed()(idxw, emb_flat)
    lin_vals = _make_sc_linear()(idxt, lin_flat)
    lint = lin_vals.reshape(F, B)

    bn = 1.0 / np.sqrt(1.0 + BN_EPS)
    w1f = W1 * (bn * g1)[None, :]
    w1p = w1f.reshape(F, D, H1).transpose(1, 0, 2).reshape(EMBED_OUT, H1)
    b1f = (b1 * bn * g1 + be1).reshape(1, H1)
    w2f = W2 * (bn * g2)[None, :]
    b2f = (b2 * bn * g2 + be2).reshape(1, H2)
    smatp = jnp.repeat(jnp.eye(D, dtype=jnp.float32), F, axis=0)
    w3c = jnp.concatenate([W3.reshape(1, H2), (lin_b + b3).reshape(1, 1)],
                          axis=1)
    return _tc_mlp(ht, lint, w1p, b1f, w2f, b2f, smatp, w3c)


# speedup vs baseline: 1.5359x; 1.4834x over previous
import functools

import jax
import jax.numpy as jnp
import numpy as np
from jax import lax
from jax.experimental import pallas as pl
from jax.experimental.pallas import tpu as pltpu
from jax.experimental.pallas import tpu_sc as plsc

F = 26
D = 16
B = 16384
VOCAB = 100000
R = F * VOCAB
BF = B * F
EMBED_OUT = F * D
H1, H2 = 128, 64
BN_EPS = 1e-5

NC, NS = 2, 16
NW = NC * NS
BW = B // NW
WROWS = F * BW


@functools.cache
def _make_sc_embed():
    mesh = plsc.VectorSubcoreMesh(core_axis_name="c", subcore_axis_name="s")

    @functools.partial(
        pl.kernel,
        mesh=mesh,
        out_type=jax.ShapeDtypeStruct((EMBED_OUT, B), jnp.float32),
        scratch_types=[
            pltpu.VMEM((WROWS,), jnp.int32),
            pltpu.VMEM((WROWS,), jnp.int32),
            pltpu.VMEM((WROWS,), jnp.int32),
            pltpu.VMEM((WROWS,), jnp.float32),
            pltpu.VMEM((WROWS,), jnp.float32),
            pltpu.SemaphoreType.DMA,
            pltpu.SemaphoreType.DMA,
            pltpu.SemaphoreType.DMA,
            pltpu.SemaphoreType.DMA,
        ],
        compiler_params=pltpu.CompilerParams(use_tc_tiling_on_sc=False),
    )
    def _sc_embed(idxw_hbm, emb_hbm, out_t,
                  idx_v, sidx_a, sidx_b, val_a, val_b,
                  gsem_a, gsem_b, wsem_a, wsem_b):
        wid = lax.axis_index("s") * NC + lax.axis_index("c")
        b0 = wid * BW
        sidx = (sidx_a, sidx_b)
        vals = (val_a, val_b)
        gsems = (gsem_a, gsem_b)
        wsems = (wsem_a, wsem_b)

        pltpu.sync_copy(idxw_hbm.at[pl.ds(wid * WROWS, WROWS)], idx_v)

        def build_and_fire(d, q):
            shift = jnp.broadcast_to(d * R, (16,)).astype(jnp.int32)

            def addloop(j, carry):
                s = pl.ds(j * 16, 16)
                sidx[q][s] = idx_v[s] + shift
                return carry

            lax.fori_loop(0, WROWS // 16, addloop, 0)
            pltpu.async_copy(emb_hbm.at[sidx[q]], vals[q], gsems[q])

        def land(d, q):
            pltpu.make_async_copy(emb_hbm.at[sidx[q]], vals[q],
                                  gsems[q]).wait()

            def wr(f, carry):
                pltpu.async_copy(vals[q].at[pl.ds(f * BW, BW)],
                                 out_t.at[d * F + f, pl.ds(b0, BW)],
                                 wsems[q])
                return carry

            lax.fori_loop(0, F, wr, 0)

        def drain(d, q):
            def wt(f, carry):
                pltpu.make_async_copy(vals[q].at[pl.ds(f * BW, BW)],
                                      out_t.at[d * F + f, pl.ds(b0, BW)],
                                      wsems[q]).wait()
                return carry

            lax.fori_loop(0, F, wt, 0)

        def even_odd(d, fn):
            p = lax.bitwise_and(d, 1)

            @pl.when(p == 0)
            def _():
                fn(d, 0)

            @pl.when(p == 1)
            def _():
                fn(d, 1)

        even_odd(jnp.int32(0), build_and_fire)

        def step(d, carry):
            @pl.when(d >= 2)
            def _():
                even_odd(d - 2, drain)

            even_odd(d, build_and_fire)
            even_odd(d - 1, land)
            return carry

        lax.fori_loop(1, D, step, 0)
        even_odd(jnp.int32(D - 2), drain)
        even_odd(jnp.int32(D - 1), land)
        even_odd(jnp.int32(D - 1), drain)

    return _sc_embed


LCHUNK = 1664
LROWS_W = BF // NW
LNCHUNK = LROWS_W // LCHUNK


@functools.cache
def _make_sc_linear():
    mesh = plsc.VectorSubcoreMesh(core_axis_name="c", subcore_axis_name="s")

    @functools.partial(
        pl.kernel,
        mesh=mesh,
        out_type=jax.ShapeDtypeStruct((BF,), jnp.float32),
        scratch_types=[
            pltpu.VMEM((LCHUNK,), jnp.int32),
            pltpu.VMEM((LCHUNK,), jnp.float32),
            pltpu.SemaphoreType.DMA,
        ],
        compiler_params=pltpu.CompilerParams(use_tc_tiling_on_sc=False),
    )
    def _sc_linear(idx_hbm, lin_hbm, out_lin, idx_v, vals_v, sem):
        wid = lax.axis_index("s") * NC + lax.axis_index("c")
        base0 = wid * LROWS_W

        def body(c, carry):
            base = base0 + c * LCHUNK
            pltpu.sync_copy(idx_hbm.at[pl.ds(base, LCHUNK)], idx_v)
            pltpu.async_copy(lin_hbm.at[idx_v], vals_v, sem).wait()
            pltpu.sync_copy(vals_v, out_lin.at[pl.ds(base, LCHUNK)])
            return carry

        lax.fori_loop(0, LNCHUNK, body, 0)

    return _sc_linear


BLK = 1024


def _tc_body(h_ref, lin_ref, w1_ref, b1_ref, w2_ref, b2_ref, smat_ref,
             w3c_ref, out_ref):
    ht = h_ref[...]
    lin = lin_ref[...]
    linear = jnp.sum(lin, axis=0)

    hh = jnp.sum(ht * ht, axis=0)
    dn = (((0,), (0,)), ((), ()))
    s = lax.dot_general(ht, smat_ref[...], dn,
                        preferred_element_type=jnp.float32)
    fm = 0.5 * (jnp.sum(s * s, axis=1) - hh)

    a1 = lax.dot_general(ht, w1_ref[...], dn,
                         preferred_element_type=jnp.float32)
    a1 = jnp.maximum(a1 + b1_ref[...], 0.0)
    a2 = jnp.dot(a1, w2_ref[...], preferred_element_type=jnp.float32)
    a2 = jnp.maximum(a2 + b2_ref[...], 0.0)
    mlp = jnp.sum(a2 * w3c_ref[...][:, :H2], axis=1) + w3c_ref[0, H2]
    out_ref[...] = linear + fm + mlp


def _tc_mlp(ht, lint, w1p, b1f, w2f, b2f, smatp, w3c):
    grid = (B // BLK,)
    return pl.pallas_call(
        _tc_body,
        grid=grid,
        in_specs=[
            pl.BlockSpec((EMBED_OUT, BLK), lambda i: (0, i)),
            pl.BlockSpec((F, BLK), lambda i: (0, i)),
            pl.BlockSpec((EMBED_OUT, H1), lambda i: (0, 0)),
            pl.BlockSpec((1, H1), lambda i: (0, 0)),
            pl.BlockSpec((H1, H2), lambda i: (0, 0)),
            pl.BlockSpec((1, H2), lambda i: (0, 0)),
            pl.BlockSpec((EMBED_OUT, D), lambda i: (0, 0)),
            pl.BlockSpec((1, H2 + 1), lambda i: (0, 0)),
        ],
        out_specs=pl.BlockSpec((BLK,), lambda i: (i,)),
        out_shape=jax.ShapeDtypeStruct((B,), jnp.float32),
    )(ht, lint, w1p, b1f, w2f, b2f, smatp, w3c)


def kernel(x, emb_table, lin_w, lin_b, W1, b1, g1, be1, W2, b2, g2, be2,
           W3, b3):
    offs = jnp.arange(F, dtype=x.dtype) * VOCAB
    idxt2 = jnp.transpose(x) + offs[:, None]
    idxt = idxt2.reshape(-1)
    idxw = idxt2.reshape(F, NW, BW).transpose(1, 0, 2).reshape(-1)
    emb_flat = jnp.concatenate([emb_table[:, d] for d in range(D)])
    lin_flat = jnp.transpose(lin_w).reshape(-1)

    ht = _make_sc_embed()(idxw, emb_flat)
    lin_vals = _make_sc_linear()(idxt, lin_flat)
    lint = lin_vals.reshape(F, B)

    bn = 1.0 / np.sqrt(1.0 + BN_EPS)
    w1f = W1 * (bn * g1)[None, :]
    w1p = w1f.reshape(F, D, H1).transpose(1, 0, 2).reshape(EMBED_OUT, H1)
    b1f = (b1 * bn * g1 + be1).reshape(1, H1)
    w2f = W2 * (bn * g2)[None, :]
    b2f = (b2 * bn * g2 + be2).reshape(1, H2)
    smatp = jnp.repeat(jnp.eye(D, dtype=jnp.float32), F, axis=0)
    w3c = jnp.concatenate([W3.reshape(1, H2), (lin_b + b3).reshape(1, 1)],
                          axis=1)
    return _tc_mlp(ht, lint, w1p, b1f, w2f, b2f, smatp, w3c)
